# Initial kernel scaffold; baseline (speedup 1.0000x reference)
import jax
import jax.numpy as jnp
from jax.experimental import pallas as pl


def kernel(x, edge_index, batch, params):
    raise NotImplementedError("write your pallas kernel here")


# speedup vs baseline: 3.2348x; 3.2348x over previous
import functools

import jax
import jax.numpy as jnp
from jax import lax
from jax.experimental import pallas as pl
from jax.experimental.pallas import tpu as pltpu
from jax.experimental.pallas import tpu_sc as plsc

N = 10000
E = 320000
D = 128
G = 64

NC = 2
NS = 16
NW = NC * NS

NPAD = 10240
EP = NW * 80 * 128
RPW = EP // NW // 128
TROWS = NPAD // NS
TCH = TROWS // 128

_sc_mesh = plsc.VectorSubcoreMesh(core_axis_name="c", subcore_axis_name="s")


@functools.partial(
    pl.kernel,
    out_type=jax.ShapeDtypeStruct((NC * NPAD, D), jnp.float32),
    mesh=_sc_mesh,
    scratch_types=[
        pltpu.VMEM((RPW, 128), jnp.int32),
        pltpu.VMEM((RPW, 128), jnp.int32),
        pltpu.VMEM((128, D), jnp.float32),
        pltpu.VMEM_SHARED((NPAD, D), jnp.float32),
        pltpu.SemaphoreType.DMA,
    ],
)
def _agg_sc(x_hbm, src_hbm, dst_hbm, out_hbm, src_v, dst_v, rows_v, acc_sh, sem):
    cid = lax.axis_index("c")
    sid = lax.axis_index("s")
    wid = cid * NS + sid

    z16 = jnp.zeros((16,), jnp.float32)

    def zrow(i, _):
        for k in range(8):
            rows_v[i, pl.ds(k * 16, 16)] = z16
        return 0

    lax.fori_loop(0, 128, zrow, 0)

    def zacc(i, _):
        pltpu.sync_copy(rows_v, acc_sh.at[pl.ds((sid * TCH + i) * 128, 128)])
        return 0

    lax.fori_loop(0, TCH, zacc, 0)

    pltpu.sync_copy(src_hbm.at[pl.ds(wid * RPW, RPW)], src_v)
    pltpu.sync_copy(dst_hbm.at[pl.ds(wid * RPW, RPW)], dst_v)
    plsc.subcore_barrier()

    def ebody(j, _):
        pltpu.async_copy(x_hbm.at[src_v.at[j]], rows_v, sem).wait()
        pltpu.sync_copy(rows_v, acc_sh.at[dst_v.at[j]], add=True)
        return 0

    lax.fori_loop(0, RPW, ebody, 0)
    plsc.subcore_barrier()

    def wout(i, _):
        row = (sid * TCH + i) * 128
        pltpu.sync_copy(acc_sh.at[pl.ds(row, 128)],
                        out_hbm.at[pl.ds(cid * NPAD + row, 128)])
        return 0

    lax.fori_loop(0, TCH, wout, 0)


@functools.partial(
    pl.kernel,
    out_type=jax.ShapeDtypeStruct((NC * 128, D), jnp.float32),
    mesh=_sc_mesh,
    scratch_types=[
        pltpu.VMEM((4, 80), jnp.int32),
        pltpu.VMEM((80, D), jnp.float32),
        pltpu.VMEM_SHARED((128, D), jnp.float32),
    ],
)
def _pool_sc(x_hbm, b_hbm, out_hbm, bidx_v, rows_v, pool_sh):
    cid = lax.axis_index("c")
    sid = lax.axis_index("s")
    wid = cid * NS + sid

    z16 = jnp.zeros((16,), jnp.float32)

    def zrow(i, _):
        for k in range(8):
            rows_v[i, pl.ds(k * 16, 16)] = z16
        return 0

    lax.fori_loop(0, 8, zrow, 0)
    pltpu.sync_copy(rows_v.at[pl.ds(0, 8)], pool_sh.at[pl.ds(sid * 8, 8)])
    pltpu.sync_copy(b_hbm.at[pl.ds(wid * 4, 4)], bidx_v)
    plsc.subcore_barrier()

    def pbody(j, _):
        pltpu.sync_copy(x_hbm.at[pl.ds((wid * 4 + j) * 80, 80)], rows_v)
        pltpu.sync_copy(rows_v, pool_sh.at[bidx_v.at[j]], add=True)
        return 0

    lax.fori_loop(0, 4, pbody, 0)
    plsc.subcore_barrier()
    pltpu.sync_copy(pool_sh.at[pl.ds(sid * 8, 8)],
                    out_hbm.at[pl.ds(cid * 128 + sid * 8, 8)])


def _mlp_body(x_ref, a_ref, w1_ref, c1_ref, w2_ref, b2_ref, o_ref):
    h = x_ref[...] + a_ref[0] + a_ref[1]
    h = jnp.dot(h, w1_ref[...], preferred_element_type=jnp.float32) + c1_ref[...]
    h = jnp.maximum(h, 0.0)
    h = jnp.dot(h, w2_ref[...], preferred_element_type=jnp.float32) + b2_ref[...]
    o_ref[...] = jnp.maximum(h, 0.0)


_MLP_BLK = 1024


def _mlp_tc(x, agg, w1, c1, w2, b2):
    return pl.pallas_call(
        _mlp_body,
        grid=(NPAD // _MLP_BLK,),
        in_specs=[
            pl.BlockSpec((_MLP_BLK, D), lambda i: (i, 0)),
            pl.BlockSpec((NC, _MLP_BLK, D), lambda i: (0, i, 0)),
            pl.BlockSpec((D, D), lambda i: (0, 0)),
            pl.BlockSpec((1, D), lambda i: (0, 0)),
            pl.BlockSpec((D, D), lambda i: (0, 0)),
            pl.BlockSpec((1, D), lambda i: (0, 0)),
        ],
        out_specs=pl.BlockSpec((_MLP_BLK, D), lambda i: (i, 0)),
        out_shape=jax.ShapeDtypeStruct((NPAD, D), jnp.float32),
    )(x, agg, w1, c1, w2, b2)


def _head_body(p_ref, w1_ref, c1_ref, w2_ref, b2_ref, o_ref):
    p = p_ref[0] + p_ref[1]
    h = jnp.dot(p, w1_ref[...], preferred_element_type=jnp.float32) + c1_ref[...]
    h = jnp.maximum(h, 0.0)
    o_ref[...] = jnp.dot(h, w2_ref[...], preferred_element_type=jnp.float32) + b2_ref[...]


def _head_tc(pool, w1, c1, w2, b2):
    return pl.pallas_call(
        _head_body,
        out_shape=jax.ShapeDtypeStruct((128, 1), jnp.float32),
    )(pool, w1, c1, w2, b2)


def kernel(x, edge_index, batch, params):
    rsq = jnp.float32(1.0) / jnp.sqrt(jnp.float32(1.0 + 1e-5))

    pad_e = EP - E
    src = jnp.concatenate(
        [edge_index[0], jnp.zeros((pad_e,), jnp.int32)]).reshape(EP // 128, 128)
    dst = jnp.concatenate(
        [edge_index[1], jnp.full((pad_e,), NPAD - 1, jnp.int32)]).reshape(EP // 128, 128)

    batch_p = jnp.concatenate(
        [batch, jnp.full((NPAD - N,), 127, jnp.int32)]).reshape(128, 80)

    h = jnp.concatenate([x, jnp.zeros((NPAD - N, D), jnp.float32)], axis=0)

    for l in range(3):
        p = params['conv%d' % l]
        g = p['gamma'] * rsq
        w1 = p['W1'] * g[None, :]
        c1 = (p['b1'] * g + p['beta']).reshape(1, D)
        agg = _agg_sc(h, src, dst).reshape(NC, NPAD, D)
        h = _mlp_tc(h, agg, w1, c1, p['W2'], p['b2'].reshape(1, D))

    pool = _pool_sc(h, batch_p).reshape(NC, 128, D)

    ph = params['lin0']
    gh = ph['gamma'] * rsq
    w1h = ph['W'] * gh[None, :]
    c1h = (ph['b'] * gh + ph['beta']).reshape(1, 64)
    out = _head_tc(pool, w1h, c1h, params['lin1']['W'],
                   params['lin1']['b'].reshape(1, 1))
    return out[:G]
